# baseline (speedup 1.0000x reference)
import functools

import jax
import jax.numpy as jnp
from jax import lax
from jax.experimental import pallas as pl
from jax.experimental.pallas import tpu as pltpu
from jax.experimental.pallas import tpu_sc as plsc

NC, NS = 2, 16
NW = NC * NS
CHUNK = 64
NBUF = 10


def _sc_gather(word_emb, ids_flat, n_tok):
  n_chunks = n_tok // CHUNK
  per_w = n_chunks // NW
  tok_w = per_w * CHUNK
  hidden = word_emb.shape[1]

  mesh = plsc.VectorSubcoreMesh(core_axis_name="c", subcore_axis_name="s")

  @functools.partial(
      pl.kernel,
      out_type=jax.ShapeDtypeStruct((n_tok, hidden), jnp.float32),
      mesh=mesh,
      scratch_types=(
          [pltpu.VMEM((tok_w,), jnp.int32),
           pltpu.VMEM((NBUF, CHUNK, hidden), jnp.float32)]
          + [pltpu.SemaphoreType.DMA] * (2 * NBUF)
      ),
  )
  def gather_kernel(word_hbm, ids_hbm, out_hbm, ids_v, rows_v, *sems):
    wid = lax.axis_index("s") * NC + lax.axis_index("c")
    base = wid * per_w
    gsems = sems[:NBUF]
    osems = sems[NBUF:]

    pltpu.sync_copy(
        ids_hbm.at[pl.ds(pl.multiple_of(wid * tok_w, 8), tok_w)], ids_v)

    def idx_view(c):
      return ids_v.at[pl.ds(pl.multiple_of(c * CHUNK, 8), CHUNK)]

    def start_gather(c, b):
      pltpu.make_async_copy(
          word_hbm.at[idx_view(c)], rows_v.at[b], gsems[b]).start()

    def wait_gather(c, b):
      pltpu.make_async_copy(
          word_hbm.at[idx_view(c)], rows_v.at[b], gsems[b]).wait()

    def out_view(c):
      return out_hbm.at[pl.ds(pl.multiple_of((base + c) * CHUNK, 8), CHUNK)]

    def start_out(c, b):
      pltpu.make_async_copy(rows_v.at[b], out_view(c), osems[b]).start()

    def wait_out(c, b):
      pltpu.make_async_copy(rows_v.at[b], out_view(c), osems[b]).wait()

    for c0 in range(NBUF - 1):
      start_gather(c0, c0)

    def body(k, carry):
      for b in range(NBUF):
        c = NBUF * k + b
        nb = (b + NBUF - 1) % NBUF

        @pl.when(c + NBUF - 1 < per_w)
        def _():
          @pl.when(c >= 1)
          def _():
            wait_out(c - 1, nb)
          start_gather(c + NBUF - 1, nb)

        wait_gather(c, b)
        start_out(c, b)
      return carry

    assert per_w % NBUF == 0
    lax.fori_loop(0, per_w // NBUF, body, 0)
    for cl in range(per_w - NBUF, per_w):
      wait_out(cl, cl % NBUF)

  return gather_kernel(word_emb, ids_flat)


def _tc_ln_kernel(w_ref, tt_ref, pos0_ref, td_ref, gamma_ref, beta_ref,
                  out_ref):
  w = w_ref[...]
  tt = tt_ref[...].astype(jnp.float32)[..., None]
  x = w + pos0_ref[...][None] + tt * td_ref[...][None]
  mu = jnp.mean(x, axis=-1, keepdims=True)
  ex2 = jnp.mean(x * x, axis=-1, keepdims=True)
  xc = x - mu
  y = xc * lax.rsqrt(ex2 - mu * mu + 1e-5)
  out_ref[...] = y * gamma_ref[...][None] + beta_ref[...][None]


def _tc_ln(rows3, tt, pos0, td, gamma2, beta2):
  bs, s, hidden = rows3.shape
  rb = 64
  return pl.pallas_call(
      _tc_ln_kernel,
      grid=(bs // rb,),
      in_specs=[
          pl.BlockSpec((rb, s, hidden), lambda i: (i, 0, 0)),
          pl.BlockSpec((rb, s), lambda i: (i, 0)),
          pl.BlockSpec((s, hidden), lambda i: (0, 0)),
          pl.BlockSpec((1, hidden), lambda i: (0, 0)),
          pl.BlockSpec((1, hidden), lambda i: (0, 0)),
          pl.BlockSpec((1, hidden), lambda i: (0, 0)),
      ],
      out_specs=pl.BlockSpec((rb, s, hidden), lambda i: (i, 0, 0)),
      out_shape=jax.ShapeDtypeStruct((bs, s, hidden), jnp.float32),
  )(rows3, tt, pos0, td, gamma2, beta2)


NSLICE = 1


def kernel(input_ids, token_type_idx, word_emb, pos_emb, type_emb,
           ln_gamma, ln_beta):
  b, s = input_ids.shape
  hidden = word_emb.shape[1]
  n_tok = b * s
  bs = b // NSLICE
  ts = bs * s

  ids_flat = input_ids.reshape(n_tok).astype(jnp.int32)
  tt = token_type_idx.astype(jnp.int32)
  pos0 = pos_emb[:s] + type_emb[0][None]
  td = (type_emb[1] - type_emb[0]).reshape(1, hidden)
  gamma2 = ln_gamma.reshape(1, hidden)
  beta2 = ln_beta.reshape(1, hidden)

  outs = []
  for i in range(NSLICE):
    rows = _sc_gather(word_emb, ids_flat[i * ts:(i + 1) * ts], ts)
    rows = rows.reshape(bs, s, hidden)
    outs.append(_tc_ln(rows, tt[i * bs:(i + 1) * bs], pos0, td,
                       gamma2, beta2))
  return jnp.concatenate(outs, axis=0)


# speedup vs baseline: 1.0607x; 1.0607x over previous
import functools

import jax
import jax.numpy as jnp
from jax import lax
from jax.experimental import pallas as pl
from jax.experimental.pallas import tpu as pltpu
from jax.experimental.pallas import tpu_sc as plsc

NC, NS = 2, 16
NW = NC * NS
CHUNK = 64
NBUF = 10


def _sc_gather(word_emb, ids_flat, n_tok):
  n_chunks = n_tok // CHUNK
  per_w = n_chunks // NW
  tok_w = per_w * CHUNK
  hidden = word_emb.shape[1]

  mesh = plsc.VectorSubcoreMesh(core_axis_name="c", subcore_axis_name="s")

  @functools.partial(
      pl.kernel,
      out_type=jax.ShapeDtypeStruct((n_tok, hidden), jnp.float32),
      mesh=mesh,
      scratch_types=(
          [pltpu.VMEM((tok_w,), jnp.int32),
           pltpu.VMEM((NBUF, CHUNK, hidden), jnp.float32)]
          + [pltpu.SemaphoreType.DMA] * (2 * NBUF)
      ),
  )
  def gather_kernel(word_hbm, ids_hbm, out_hbm, ids_v, rows_v, *sems):
    wid = lax.axis_index("s") * NC + lax.axis_index("c")
    base = wid * per_w
    gsems = sems[:NBUF]
    osems = sems[NBUF:]

    pltpu.sync_copy(
        ids_hbm.at[pl.ds(pl.multiple_of(wid * tok_w, 8), tok_w)], ids_v)

    def idx_view(c):
      return ids_v.at[pl.ds(pl.multiple_of(c * CHUNK, 8), CHUNK)]

    def start_gather(c, b):
      pltpu.make_async_copy(
          word_hbm.at[idx_view(c)], rows_v.at[b], gsems[b]).start()

    def wait_gather(c, b):
      pltpu.make_async_copy(
          word_hbm.at[idx_view(c)], rows_v.at[b], gsems[b]).wait()

    def out_view(c):
      return out_hbm.at[pl.ds(pl.multiple_of((base + c) * CHUNK, 8), CHUNK)]

    def start_out(c, b):
      pltpu.make_async_copy(rows_v.at[b], out_view(c), osems[b]).start()

    def wait_out(c, b):
      pltpu.make_async_copy(rows_v.at[b], out_view(c), osems[b]).wait()

    for c0 in range(NBUF - 1):
      start_gather(c0, c0)

    def body(k, carry):
      for b in range(NBUF):
        c = NBUF * k + b
        nb = (b + NBUF - 1) % NBUF

        @pl.when(c + NBUF - 1 < per_w)
        def _():
          @pl.when(c >= 1)
          def _():
            wait_out(c - 1, nb)
          start_gather(c + NBUF - 1, nb)

        wait_gather(c, b)
        start_out(c, b)
      return carry

    assert per_w % NBUF == 0
    lax.fori_loop(0, per_w // NBUF, body, 0)
    for cl in range(per_w - NBUF, per_w):
      wait_out(cl, cl % NBUF)

  return gather_kernel(word_emb, ids_flat)


def _tc_ln_kernel(w_ref, tt_ref, pos0_ref, td_ref, gamma_ref, beta_ref,
                  out_ref):
  w = w_ref[...]
  tt = tt_ref[...].astype(jnp.float32)[..., None]
  x = w + pos0_ref[...][None] + tt * td_ref[...][None]
  mu = jnp.mean(x, axis=-1, keepdims=True)
  xc = x - mu
  var = jnp.mean(xc * xc, axis=-1, keepdims=True)
  y = xc * lax.rsqrt(var + 1e-5)
  out_ref[...] = y * gamma_ref[...][None] + beta_ref[...][None]


def _tc_ln(rows3, tt, pos0, td, gamma2, beta2):
  bs, s, hidden = rows3.shape
  rb = 64
  return pl.pallas_call(
      _tc_ln_kernel,
      grid=(bs // rb,),
      in_specs=[
          pl.BlockSpec((rb, s, hidden), lambda i: (i, 0, 0)),
          pl.BlockSpec((rb, s), lambda i: (i, 0)),
          pl.BlockSpec((s, hidden), lambda i: (0, 0)),
          pl.BlockSpec((1, hidden), lambda i: (0, 0)),
          pl.BlockSpec((1, hidden), lambda i: (0, 0)),
          pl.BlockSpec((1, hidden), lambda i: (0, 0)),
      ],
      out_specs=pl.BlockSpec((rb, s, hidden), lambda i: (i, 0, 0)),
      out_shape=jax.ShapeDtypeStruct((bs, s, hidden), jnp.float32),
  )(rows3, tt, pos0, td, gamma2, beta2)


NSLICE = 1


def kernel(input_ids, token_type_idx, word_emb, pos_emb, type_emb,
           ln_gamma, ln_beta):
  b, s = input_ids.shape
  hidden = word_emb.shape[1]
  n_tok = b * s
  bs = b // NSLICE
  ts = bs * s

  ids_flat = input_ids.reshape(n_tok).astype(jnp.int32)
  tt = token_type_idx.astype(jnp.int32)
  pos0 = pos_emb[:s] + type_emb[0][None]
  td = (type_emb[1] - type_emb[0]).reshape(1, hidden)
  gamma2 = ln_gamma.reshape(1, hidden)
  beta2 = ln_beta.reshape(1, hidden)

  outs = []
  for i in range(NSLICE):
    rows = _sc_gather(word_emb, ids_flat[i * ts:(i + 1) * ts], ts)
    rows = rows.reshape(bs, s, hidden)
    outs.append(_tc_ln(rows, tt[i * bs:(i + 1) * bs], pos0, td,
                       gamma2, beta2))
  return jnp.concatenate(outs, axis=0)


# speedup vs baseline: 1.0620x; 1.0013x over previous
import functools

import jax
import jax.numpy as jnp
from jax import lax
from jax.experimental import pallas as pl
from jax.experimental.pallas import tpu as pltpu
from jax.experimental.pallas import tpu_sc as plsc

NC, NS = 2, 16
NW = NC * NS
CHUNK = 64
NBUF = 10


def _sc_gather(word_emb, ids_flat, n_tok):
  n_chunks = n_tok // CHUNK
  per_w = n_chunks // NW
  tok_w = per_w * CHUNK
  hidden = word_emb.shape[1]

  mesh = plsc.VectorSubcoreMesh(core_axis_name="c", subcore_axis_name="s")

  @functools.partial(
      pl.kernel,
      out_type=jax.ShapeDtypeStruct((n_tok, hidden), jnp.float32),
      mesh=mesh,
      scratch_types=(
          [pltpu.VMEM((tok_w,), jnp.int32),
           pltpu.VMEM((NBUF, CHUNK, hidden), jnp.float32)]
          + [pltpu.SemaphoreType.DMA] * (2 * NBUF)
      ),
  )
  def gather_kernel(word_hbm, ids_hbm, out_hbm, ids_v, rows_v, *sems):
    wid = lax.axis_index("s") * NC + lax.axis_index("c")
    base = wid * per_w
    gsems = sems[:NBUF]
    osems = sems[NBUF:]

    pltpu.sync_copy(
        ids_hbm.at[pl.ds(pl.multiple_of(wid * tok_w, 8), tok_w)], ids_v)

    def idx_view(c):
      return ids_v.at[pl.ds(pl.multiple_of(c * CHUNK, 8), CHUNK)]

    def start_gather(c, b):
      pltpu.make_async_copy(
          word_hbm.at[idx_view(c)], rows_v.at[b], gsems[b]).start()

    def wait_gather(c, b):
      pltpu.make_async_copy(
          word_hbm.at[idx_view(c)], rows_v.at[b], gsems[b]).wait()

    def out_view(c):
      return out_hbm.at[pl.ds(pl.multiple_of((base + c) * CHUNK, 8), CHUNK)]

    def start_out(c, b):
      pltpu.make_async_copy(rows_v.at[b], out_view(c), osems[b]).start()

    def wait_out(c, b):
      pltpu.make_async_copy(rows_v.at[b], out_view(c), osems[b]).wait()

    for c0 in range(NBUF - 1):
      start_gather(c0, c0)

    def body(k, carry):
      for b in range(NBUF):
        c = NBUF * k + b
        nb = (b + NBUF - 1) % NBUF

        @pl.when(c + NBUF - 1 < per_w)
        def _():
          @pl.when(c >= 1)
          def _():
            wait_out(c - 1, nb)
          start_gather(c + NBUF - 1, nb)

        wait_gather(c, b)
        start_out(c, b)
      return carry

    assert per_w % NBUF == 0
    lax.fori_loop(0, per_w // NBUF, body, 0)
    for cl in range(per_w - NBUF, per_w):
      wait_out(cl, cl % NBUF)

  return gather_kernel(word_emb, ids_flat)


def _tc_ln_kernel(w_ref, tt_ref, pos0_ref, td_ref, gamma_ref, beta_ref,
                  out_ref):
  w = w_ref[...]
  tt = tt_ref[...].astype(jnp.float32)[..., None]
  x = w + pos0_ref[...][None] + tt * td_ref[...][None]
  mu = jnp.mean(x, axis=-1, keepdims=True)
  xc = x - mu
  var = jnp.mean(xc * xc, axis=-1, keepdims=True)
  y = xc * lax.rsqrt(var + 1e-5)
  out_ref[...] = y * gamma_ref[...][None] + beta_ref[...][None]


def _tc_ln(rows3, tt, pos0, td, gamma2, beta2):
  bs, s, hidden = rows3.shape
  rb = 64
  return pl.pallas_call(
      _tc_ln_kernel,
      grid=(bs // rb,),
      in_specs=[
          pl.BlockSpec((rb, s, hidden), lambda i: (i, 0, 0)),
          pl.BlockSpec((rb, s), lambda i: (i, 0)),
          pl.BlockSpec((s, hidden), lambda i: (0, 0)),
          pl.BlockSpec((1, hidden), lambda i: (0, 0)),
          pl.BlockSpec((1, hidden), lambda i: (0, 0)),
          pl.BlockSpec((1, hidden), lambda i: (0, 0)),
      ],
      out_specs=pl.BlockSpec((rb, s, hidden), lambda i: (i, 0, 0)),
      out_shape=jax.ShapeDtypeStruct((bs, s, hidden), jnp.float32),
  )(rows3, tt, pos0, td, gamma2, beta2)


def kernel(input_ids, token_type_idx, word_emb, pos_emb, type_emb,
           ln_gamma, ln_beta):
  b, s = input_ids.shape
  hidden = word_emb.shape[1]
  n_tok = b * s

  ids_flat = input_ids.reshape(n_tok).astype(jnp.int32)
  tt = token_type_idx.astype(jnp.int32)
  pos0 = pos_emb[:s] + type_emb[0][None]
  td = (type_emb[1] - type_emb[0]).reshape(1, hidden)
  gamma2 = ln_gamma.reshape(1, hidden)
  beta2 = ln_beta.reshape(1, hidden)

  rows = _sc_gather(word_emb, ids_flat, n_tok).reshape(b, s, hidden)
  return _tc_ln(rows, tt, pos0, td, gamma2, beta2)
